# baseline (speedup 1.0000x reference)
import functools

import jax
import jax.numpy as jnp
from jax import lax
from jax.experimental import pallas as pl
from jax.experimental.pallas import tpu as pltpu
from jax.experimental.pallas import tpu_sc as plsc

N = 10000
NP = 10240
F_IN = 128
HID = 256
E = 160000
EPT = 10240
NCH = EPT // 128
STRIPE = NP // 16

_mesh = plsc.VectorSubcoreMesh(core_axis_name="c", subcore_axis_name="s",
                               num_cores=2, num_subcores=16)


def _tc_logits_body(x_ref, w_ref, alar_ref, eler_ref):
    xb = x_ref[...]
    for r in range(2):
        cl = jnp.dot(w_ref[r], alar_ref[r], preferred_element_type=jnp.float32)
        elr = jnp.dot(xb, cl, preferred_element_type=jnp.float32)
        eler_ref[0, r] = elr[:, 0]
        eler_ref[1, r] = elr[:, 1]


def _tc_logits(x, W0, W1, al0, al1, ar0, ar1):
    R = 256
    grid = (NP // R,)
    return pl.pallas_call(
        _tc_logits_body,
        grid=grid,
        in_specs=[
            pl.BlockSpec((R, F_IN), lambda i: (i, 0)),
            pl.BlockSpec((2, F_IN, HID), lambda i: (0, 0, 0)),
            pl.BlockSpec((2, HID, 2), lambda i: (0, 0, 0)),
        ],
        out_specs=pl.BlockSpec((2, 2, R), lambda i: (0, 0, i)),
        out_shape=jax.ShapeDtypeStruct((2, 2, NP), jnp.float32),
    )(x, jnp.stack([W0, W1]),
      jnp.stack([jnp.stack([al0, ar0], 1), jnp.stack([al1, ar1], 1)]))


def _tc_mid_body(*refs):
    a = refs[:4]
    s_ref, b_ref, w1_ref, w2_ref, alar_ref, feat_ref, eler_ref = refs[4:]
    s0 = jnp.maximum(s_ref[0], 1e-30)[:, None]
    s1 = jnp.maximum(s_ref[1], 1e-30)[:, None]
    bm = 0.5 * (b_ref[0] + b_ref[1])
    agg0 = jnp.concatenate([a[0][0, 0], a[1][0, 0]], axis=1) / s0
    agg1 = jnp.concatenate([a[2][0, 0], a[3][0, 0]], axis=1) / s1
    h = 0.5 * (jnp.dot(agg0, w1_ref[0], preferred_element_type=jnp.float32)
               + jnp.dot(agg1, w1_ref[1], preferred_element_type=jnp.float32))
    hb = jax.nn.relu(h + bm[None, :])
    for r in range(2):
        f = jnp.dot(hb, w2_ref[r], preferred_element_type=jnp.float32)
        feat_ref[r, 0] = f[:, :64]
        feat_ref[r, 1] = f[:, 64:]
        elr = jnp.dot(f, alar_ref[r], preferred_element_type=jnp.float32)
        eler_ref[0, r] = elr[:, 0]
        eler_ref[1, r] = elr[:, 1]


def _tc_mid(acc1, s1, b0, b1, W1_0, W1_1, W0, W1, al0, al1, ar0, ar1):
    R = 256
    grid = (NP // R,)
    views = [pl.BlockSpec((1, 1, R, 64), functools.partial(
        lambda i, r, q: (r, q, i, 0), r=r, q=q))
        for r in (0, 1) for q in range(2)]
    return pl.pallas_call(
        _tc_mid_body,
        grid=grid,
        in_specs=views + [
            pl.BlockSpec((2, R), lambda i: (0, i)),
            pl.BlockSpec((2, HID), lambda i: (0, 0)),
            pl.BlockSpec((2, F_IN, HID), lambda i: (0, 0, 0)),
            pl.BlockSpec((2, HID, 128), lambda i: (0, 0, 0)),
            pl.BlockSpec((2, 128, 2), lambda i: (0, 0, 0)),
        ],
        out_specs=[
            pl.BlockSpec((2, 2, R, 64), lambda i: (0, 0, i, 0)),
            pl.BlockSpec((2, 2, R), lambda i: (0, 0, i)),
        ],
        out_shape=[
            jax.ShapeDtypeStruct((2, 2, NP, 64), jnp.float32),
            jax.ShapeDtypeStruct((2, 2, NP), jnp.float32),
        ],
    )(*([acc1] * 4), s1, jnp.stack([b0, b1]), jnp.stack([W1_0, W1_1]),
      jnp.stack([W0, W1]),
      jnp.stack([jnp.stack([al0, ar0], 1), jnp.stack([al1, ar1], 1)]))


def _tc_out_body(a00, a01, a10, a11, s_ref, b_ref, wl_ref, blin_ref, uv_ref):
    s0 = jnp.maximum(s_ref[0], 1e-30)[:, None]
    s1 = jnp.maximum(s_ref[1], 1e-30)[:, None]
    bm = 0.5 * (b_ref[0] + b_ref[1])
    hh = jnp.concatenate(
        [0.5 * (a00[0, 0] / s0 + a10[0, 0] / s1) + bm[None, :64],
         0.5 * (a01[0, 0] / s0 + a11[0, 0] / s1) + bm[None, 64:]], axis=1)
    r = jax.nn.relu(hh)
    uvb = jnp.dot(r, wl_ref[...], preferred_element_type=jnp.float32)
    uv_ref[0] = uvb[:, 0] + blin_ref[0]
    uv_ref[1] = uvb[:, 1]


def _tc_out(acc2, s2, b0, b1, W_lin, b_lin):
    R = 256
    grid = (NP // R,)
    wl = jnp.stack([W_lin[:128, 0], W_lin[128:, 0]], axis=1)
    views = [pl.BlockSpec((1, 1, R, 64), functools.partial(
        lambda i, r, q: (r, q, i, 0), r=r, q=q))
        for r in (0, 1) for q in (0, 1)]
    return pl.pallas_call(
        _tc_out_body,
        grid=grid,
        in_specs=views + [
            pl.BlockSpec((2, R), lambda i: (0, i)),
            pl.BlockSpec((2, 128), lambda i: (0, 0)),
            pl.BlockSpec((128, 2), lambda i: (0, 0)),
            pl.BlockSpec(memory_space=pltpu.SMEM),
        ],
        out_specs=pl.BlockSpec((2, R), lambda i: (0, i)),
        out_shape=jax.ShapeDtypeStruct((2, NP), jnp.float32),
    )(acc2, acc2, acc2, acc2, s2, jnp.stack([b0, b1]), wl, b_lin)


def _sc_layer_body(Q, featQ, eler, srcb, dstb, acc_out, s_out,
                   el_v, er_v, sb_v, db_v, ex_v, gbufs,
                   zs_v, gsems, ssems, s_sh, acc_sh):
    c = lax.axis_index("c")
    s = lax.axis_index("s")
    pltpu.sync_copy(eler.at[0, c], el_v)
    pltpu.sync_copy(eler.at[1, c], er_v)
    pltpu.sync_copy(srcb.at[c, s], sb_v)
    pltpu.sync_copy(dstb.at[c, s], db_v)

    zv = jnp.zeros((16,), jnp.float32)

    def zsrow(r, carry):
        zs_v[pl.ds(r * 16, 16)] = zv
        return carry
    lax.fori_loop(0, STRIPE // 16, zsrow, 0)
    pltpu.sync_copy(zs_v, s_sh.at[pl.ds(s * STRIPE, STRIPE)])

    @plsc.parallel_loop(0, EPT // 16, unroll=4)
    def att(g):
        si = sb_v[g // 8, pl.ds((g % 8) * 16, 16)]
        di = db_v[g // 8, pl.ds((g % 8) * 16, 16)]
        e = plsc.load_gather(el_v, [si]) + plsc.load_gather(er_v, [di])
        e = jnp.where(e > 0, e, 0.2 * e)
        ex_v[g // 8, pl.ds((g % 8) * 16, 16)] = jnp.exp(e)
    plsc.subcore_barrier()

    def satt(k, carry):
        for b in range(8):
            pltpu.async_copy(ex_v.at[k * 8 + b], s_sh.at[db_v.at[k * 8 + b]],
                             ssems.at[0], add=True)
        for b in range(8):
            pltpu.make_async_copy(ex_v.at[k * 8 + b],
                                  s_sh.at[db_v.at[k * 8 + b]],
                                  ssems.at[0]).wait()
        return carry
    lax.fori_loop(0, NCH // 8, satt, 0)
    plsc.subcore_barrier()

    pltpu.sync_copy(s_sh.at[pl.ds(s * STRIPE, STRIPE)],
                    s_out.at[c, pl.ds(s * STRIPE, STRIPE)])

    def gather_chunk(q, j, b):
        return pltpu.async_copy(featQ.at[c, q].at[sb_v.at[j]], gbufs.at[b],
                                gsems.at[b])

    def scatter_chunk(j, b):
        return pltpu.async_copy(gbufs.at[b], acc_sh.at[db_v.at[j]],
                                ssems.at[b], add=True)

    for q in range(Q):
        def zgrow(r, carry):
            for c8 in range(4):
                gbufs[0, r, pl.ds(c8 * 16, 16)] = zv
            return carry
        lax.fori_loop(0, 128, zgrow, 0)
        for k in range(STRIPE // 128):
            pltpu.sync_copy(gbufs.at[0], acc_sh.at[pl.ds(s * STRIPE + k * 128, 128)])
        plsc.subcore_barrier()

        gather_chunk(q, 0, 0)
        gather_chunk(q, 1, 1)

        def quad(k, carry):
            for b in range(4):
                m = k * 4 + b
                pltpu.make_async_copy(featQ.at[c, q].at[sb_v.at[m]],
                                      gbufs.at[b], gsems.at[b]).wait()

                @plsc.parallel_loop(0, 128, unroll=8)
                def scale(e):
                    av = plsc.load_gather(
                        ex_v, [jnp.full((16,), m, jnp.int32),
                               jnp.full((16,), e, jnp.int32)])
                    for c8 in range(4):
                        gbufs[b, e, pl.ds(c8 * 16, 16)] = (
                            gbufs[b, e, pl.ds(c8 * 16, 16)] * av)
                scatter_chunk(m, b)
                bn = (b + 2) % 4

                def drain_prev():
                    pltpu.make_async_copy(gbufs.at[bn],
                                          acc_sh.at[db_v.at[m - 2]],
                                          ssems.at[bn]).wait()
                if b >= 2:
                    drain_prev()
                else:
                    pl.when(k > 0)(drain_prev)
                gather_chunk(q, m + 2, bn)
            return carry
        lax.fori_loop(0, NCH // 4, quad, 0)

        pltpu.make_async_copy(gbufs.at[(NCH - 2) % 4],
                              acc_sh.at[db_v.at[NCH - 2]],
                              ssems.at[(NCH - 2) % 4]).wait()
        pltpu.make_async_copy(gbufs.at[(NCH - 1) % 4],
                              acc_sh.at[db_v.at[NCH - 1]],
                              ssems.at[(NCH - 1) % 4]).wait()
        pltpu.make_async_copy(featQ.at[c, q].at[sb_v.at[NCH]],
                              gbufs.at[NCH % 4], gsems.at[NCH % 4]).wait()
        pltpu.make_async_copy(featQ.at[c, q].at[sb_v.at[NCH + 1]],
                              gbufs.at[(NCH + 1) % 4],
                              gsems.at[(NCH + 1) % 4]).wait()
        plsc.subcore_barrier()

        pltpu.sync_copy(acc_sh.at[pl.ds(s * STRIPE, STRIPE)],
                        acc_out.at[c, q, pl.ds(s * STRIPE, STRIPE)])
        if q + 1 < Q:
            plsc.subcore_barrier()


def _sc_layer(Q, featQ, eler, srcb, dstb):
    return pl.kernel(
        functools.partial(_sc_layer_body, Q),
        out_type=(
            jax.ShapeDtypeStruct((2, Q, NP, 64), jnp.float32),
            jax.ShapeDtypeStruct((2, NP), jnp.float32),
        ),
        mesh=_mesh,
        compiler_params=pltpu.CompilerParams(needs_layout_passes=False,
                                             use_tc_tiling_on_sc=False),
        scratch_types=[
            pltpu.VMEM((NP,), jnp.float32),
            pltpu.VMEM((NP,), jnp.float32),
            pltpu.VMEM((NCH + 2, 128), jnp.int32),
            pltpu.VMEM((NCH, 128), jnp.int32),
            pltpu.VMEM((NCH, 128), jnp.float32),
            pltpu.VMEM((4, 128, 64), jnp.float32),
            pltpu.VMEM((STRIPE,), jnp.float32),
            pltpu.SemaphoreType.DMA((4,)),
            pltpu.SemaphoreType.DMA((4,)),
            pltpu.VMEM_SHARED((NP,), jnp.float32),
            pltpu.VMEM_SHARED((NP, 64), jnp.float32),
        ],
    )(featQ, eler, srcb, dstb)


TPW = 10320


def _sc_readout_body(uv, aif, bif, o_hbm, u_v, v_v, ai_v, bi_v, o_v):
    c = lax.axis_index("c")
    s = lax.axis_index("s")
    w = c * 16 + s
    pltpu.sync_copy(uv.at[0], u_v)
    pltpu.sync_copy(uv.at[1], v_v)
    pltpu.sync_copy(aif.at[w], ai_v)
    pltpu.sync_copy(bif.at[w], bi_v)

    def body(g, carry):
        ai = ai_v[pl.ds(g * 16, 16)]
        bi = bi_v[pl.ds(g * 16, 16)]
        t = plsc.load_gather(u_v, [ai]) + plsc.load_gather(v_v, [bi])
        o_v[pl.ds(g * 16, 16)] = 1.0 / (1.0 + jnp.exp(-t))
        return carry
    lax.fori_loop(0, TPW // 16, body, 0)
    pltpu.sync_copy(o_v, o_hbm.at[w])


def _sc_readout(uv, aif, bif):
    return pl.kernel(
        _sc_readout_body,
        out_type=jax.ShapeDtypeStruct((32, TPW), jnp.float32),
        mesh=_mesh,
        compiler_params=pltpu.CompilerParams(needs_layout_passes=False),
        scratch_types=[
            pltpu.VMEM((NP,), jnp.float32),
            pltpu.VMEM((NP,), jnp.float32),
            pltpu.VMEM((TPW,), jnp.int32),
            pltpu.VMEM((TPW,), jnp.int32),
            pltpu.VMEM((TPW,), jnp.float32),
        ],
    )(uv, aif, bif)


def _prep_edges(ei):
    order = jnp.argsort(ei[0])
    src = ei[0][order].reshape(16, E // 16)
    dst = ei[1][order].reshape(16, E // 16)
    pad = EPT - E // 16
    src = jnp.pad(src, ((0, 0), (0, pad)))
    dst = jnp.pad(dst, ((0, 0), (0, pad)), constant_values=NP - 1)
    return src, dst


def kernel(x, ei0, ei1, n_pairs, W1_0, al1_0, ar1_0, b1_0, W1_1, al1_1,
           ar1_1, b1_1, W2_0, al2_0, ar2_0, b2_0, W2_1, al2_1, ar2_1, b2_1,
           W_lin, b_lin):
    x_pad = jnp.pad(x, ((0, NP - N), (0, 0)))

    s0, d0 = _prep_edges(ei0)
    s1, d1 = _prep_edges(ei1)
    srcb = jnp.pad(jnp.stack([s0, s1]).reshape(2, 16, NCH, 128),
                   ((0, 0), (0, 0), (0, 2), (0, 0)))
    dstb = jnp.stack([d0, d1]).reshape(2, 16, NCH, 128)

    xQ = jnp.stack([x_pad[:, :64], x_pad[:, 64:]])
    featX = jnp.stack([xQ, xQ])
    eler1 = _tc_logits(x_pad, W1_0, W1_1, al1_0, al1_1, ar1_0, ar1_1)
    acc1, sden1 = _sc_layer(2, featX, eler1, srcb, dstb)
    feat2, eler2 = _tc_mid(acc1, sden1, b1_0, b1_1, W1_0, W1_1, W2_0, W2_1,
                           al2_0, al2_1, ar2_0, ar2_1)
    acc2, sden2 = _sc_layer(2, feat2, eler2, srcb, dstb)
    uv = _tc_out(acc2, sden2, b2_0, b2_1, W_lin, b_lin)

    a_idx = jnp.concatenate([ei0[0], ei1[0], n_pairs[:, 0]])
    b_idx = jnp.concatenate([ei0[1], ei1[1], n_pairs[:, 1]])
    npad = 32 * TPW - a_idx.shape[0]
    aif = jnp.pad(a_idx, (0, npad)).reshape(32, TPW)
    bif = jnp.pad(b_idx, (0, npad)).reshape(32, TPW)

    out = _sc_readout(uv, aif, bif)
    return out.reshape(-1)[: 2 * E + n_pairs.shape[0], None]


# speedup vs baseline: 1.3554x; 1.3554x over previous
import functools

import jax
import jax.numpy as jnp
from jax import lax
from jax.experimental import pallas as pl
from jax.experimental.pallas import tpu as pltpu
from jax.experimental.pallas import tpu_sc as plsc

N = 10000
NP = 10240
F_IN = 128
HID = 256
E = 160000
EPT = 10240
CH = 64
NCH = EPT // CH
STRIPE = NP // 16
RD = 8

_mesh = plsc.VectorSubcoreMesh(core_axis_name="c", subcore_axis_name="s",
                               num_cores=2, num_subcores=16)


def _tc_logits_body(x_ref, w_ref, alar_ref, eler_ref):
    xb = x_ref[...]
    for r in range(2):
        cl = jnp.dot(w_ref[r], alar_ref[r], preferred_element_type=jnp.float32)
        elr = jnp.dot(xb, cl, preferred_element_type=jnp.float32)
        eler_ref[0, r] = elr[:, 0]
        eler_ref[1, r] = elr[:, 1]


def _tc_logits(x, W0, W1, al0, al1, ar0, ar1):
    R = 256
    grid = (NP // R,)
    return pl.pallas_call(
        _tc_logits_body,
        grid=grid,
        in_specs=[
            pl.BlockSpec((R, F_IN), lambda i: (i, 0)),
            pl.BlockSpec((2, F_IN, HID), lambda i: (0, 0, 0)),
            pl.BlockSpec((2, HID, 2), lambda i: (0, 0, 0)),
        ],
        out_specs=pl.BlockSpec((2, 2, R), lambda i: (0, 0, i)),
        out_shape=jax.ShapeDtypeStruct((2, 2, NP), jnp.float32),
    )(x, jnp.stack([W0, W1]),
      jnp.stack([jnp.stack([al0, ar0], 1), jnp.stack([al1, ar1], 1)]))


def _tc_mid_body(*refs):
    a = refs[:4]
    s_ref, b_ref, w1_ref, w2_ref, alar_ref, feat_ref, eler_ref = refs[4:]
    s0 = jnp.maximum(s_ref[0], 1e-30)[:, None]
    s1 = jnp.maximum(s_ref[1], 1e-30)[:, None]
    bm = 0.5 * (b_ref[0] + b_ref[1])
    agg0 = jnp.concatenate([a[0][0, 0], a[1][0, 0]], axis=1) / s0
    agg1 = jnp.concatenate([a[2][0, 0], a[3][0, 0]], axis=1) / s1
    h = 0.5 * (jnp.dot(agg0, w1_ref[0], preferred_element_type=jnp.float32)
               + jnp.dot(agg1, w1_ref[1], preferred_element_type=jnp.float32))
    hb = jax.nn.relu(h + bm[None, :])
    for r in range(2):
        f = jnp.dot(hb, w2_ref[r], preferred_element_type=jnp.float32)
        feat_ref[r, 0] = f[:, :64]
        feat_ref[r, 1] = f[:, 64:]
        elr = jnp.dot(f, alar_ref[r], preferred_element_type=jnp.float32)
        eler_ref[0, r] = elr[:, 0]
        eler_ref[1, r] = elr[:, 1]


def _tc_mid(acc1, s1, b0, b1, W1_0, W1_1, W0, W1, al0, al1, ar0, ar1):
    R = 256
    grid = (NP // R,)
    views = [pl.BlockSpec((1, 1, R, 64), functools.partial(
        lambda i, r, q: (r, q, i, 0), r=r, q=q))
        for r in (0, 1) for q in range(2)]
    return pl.pallas_call(
        _tc_mid_body,
        grid=grid,
        in_specs=views + [
            pl.BlockSpec((2, R), lambda i: (0, i)),
            pl.BlockSpec((2, HID), lambda i: (0, 0)),
            pl.BlockSpec((2, F_IN, HID), lambda i: (0, 0, 0)),
            pl.BlockSpec((2, HID, 128), lambda i: (0, 0, 0)),
            pl.BlockSpec((2, 128, 2), lambda i: (0, 0, 0)),
        ],
        out_specs=[
            pl.BlockSpec((2, 2, R, 64), lambda i: (0, 0, i, 0)),
            pl.BlockSpec((2, 2, R), lambda i: (0, 0, i)),
        ],
        out_shape=[
            jax.ShapeDtypeStruct((2, 2, NP, 64), jnp.float32),
            jax.ShapeDtypeStruct((2, 2, NP), jnp.float32),
        ],
    )(*([acc1] * 4), s1, jnp.stack([b0, b1]), jnp.stack([W1_0, W1_1]),
      jnp.stack([W0, W1]),
      jnp.stack([jnp.stack([al0, ar0], 1), jnp.stack([al1, ar1], 1)]))


def _tc_out_body(a00, a01, a10, a11, s_ref, b_ref, wl_ref, blin_ref, uv_ref):
    s0 = jnp.maximum(s_ref[0], 1e-30)[:, None]
    s1 = jnp.maximum(s_ref[1], 1e-30)[:, None]
    bm = 0.5 * (b_ref[0] + b_ref[1])
    hh = jnp.concatenate(
        [0.5 * (a00[0, 0] / s0 + a10[0, 0] / s1) + bm[None, :64],
         0.5 * (a01[0, 0] / s0 + a11[0, 0] / s1) + bm[None, 64:]], axis=1)
    r = jax.nn.relu(hh)
    uvb = jnp.dot(r, wl_ref[...], preferred_element_type=jnp.float32)
    uv_ref[0] = uvb[:, 0] + blin_ref[0]
    uv_ref[1] = uvb[:, 1]


def _tc_out(acc2, s2, b0, b1, W_lin, b_lin):
    R = 256
    grid = (NP // R,)
    wl = jnp.stack([W_lin[:128, 0], W_lin[128:, 0]], axis=1)
    views = [pl.BlockSpec((1, 1, R, 64), functools.partial(
        lambda i, r, q: (r, q, i, 0), r=r, q=q))
        for r in (0, 1) for q in (0, 1)]
    return pl.pallas_call(
        _tc_out_body,
        grid=grid,
        in_specs=views + [
            pl.BlockSpec((2, R), lambda i: (0, i)),
            pl.BlockSpec((2, 128), lambda i: (0, 0)),
            pl.BlockSpec((128, 2), lambda i: (0, 0)),
            pl.BlockSpec(memory_space=pltpu.SMEM),
        ],
        out_specs=pl.BlockSpec((2, R), lambda i: (0, i)),
        out_shape=jax.ShapeDtypeStruct((2, NP), jnp.float32),
    )(acc2, acc2, acc2, acc2, s2, jnp.stack([b0, b1]), wl, b_lin)


def _sc_layer_body(Q, featQ, eler, srcb, dstb, acc_out, s_out,
                   el_v, er_v, sb_v, db_v, ex_v, gbufs,
                   zs_v, gsems, ssems, s_sh, acc_sh):
    c = lax.axis_index("c")
    s = lax.axis_index("s")
    pltpu.sync_copy(eler.at[0, c], el_v)
    pltpu.sync_copy(eler.at[1, c], er_v)
    pltpu.sync_copy(srcb.at[c, s], sb_v)
    pltpu.sync_copy(dstb.at[c, s], db_v)

    zv = jnp.zeros((16,), jnp.float32)

    def zsrow(r, carry):
        zs_v[pl.ds(r * 16, 16)] = zv
        return carry
    lax.fori_loop(0, STRIPE // 16, zsrow, 0)
    pltpu.sync_copy(zs_v, s_sh.at[pl.ds(s * STRIPE, STRIPE)])

    NG = CH // 16
    @plsc.parallel_loop(0, EPT // 16, unroll=4)
    def att(g):
        si = sb_v[g // NG, pl.ds((g % NG) * 16, 16)]
        di = db_v[g // NG, pl.ds((g % NG) * 16, 16)]
        e = plsc.load_gather(el_v, [si]) + plsc.load_gather(er_v, [di])
        e = jnp.where(e > 0, e, 0.2 * e)
        ex_v[g // NG, pl.ds((g % NG) * 16, 16)] = jnp.exp(e)
    plsc.subcore_barrier()

    def satt(k, carry):
        for b in range(8):
            pltpu.async_copy(ex_v.at[k * 8 + b], s_sh.at[db_v.at[k * 8 + b]],
                             ssems.at[0], add=True)
        for b in range(8):
            pltpu.make_async_copy(ex_v.at[k * 8 + b],
                                  s_sh.at[db_v.at[k * 8 + b]],
                                  ssems.at[0]).wait()
        return carry
    lax.fori_loop(0, NCH // 8, satt, 0)
    plsc.subcore_barrier()

    pltpu.sync_copy(s_sh.at[pl.ds(s * STRIPE, STRIPE)],
                    s_out.at[c, pl.ds(s * STRIPE, STRIPE)])

    def gather_chunk(q, j, b):
        return pltpu.async_copy(featQ.at[c, q].at[sb_v.at[j]], gbufs.at[b],
                                gsems.at[b])

    def scatter_chunk(j, b):
        return pltpu.async_copy(gbufs.at[b], acc_sh.at[db_v.at[j]],
                                ssems.at[b], add=True)

    for q in range(Q):
        def zgrow(r, carry):
            for c8 in range(4):
                gbufs[0, r, pl.ds(c8 * 16, 16)] = zv
            return carry
        lax.fori_loop(0, CH, zgrow, 0)
        for k in range(STRIPE // CH):
            pltpu.sync_copy(gbufs.at[0], acc_sh.at[pl.ds(s * STRIPE + k * CH, CH)])
        plsc.subcore_barrier()

        for b0 in range(RD - 2):
            gather_chunk(q, b0, b0)

        def octo(k, carry):
            for b in range(RD):
                m = k * RD + b
                pltpu.make_async_copy(featQ.at[c, q].at[sb_v.at[m]],
                                      gbufs.at[b], gsems.at[b]).wait()

                @plsc.parallel_loop(0, CH, unroll=8)
                def scale(e):
                    av = plsc.load_gather(
                        ex_v, [jnp.full((16,), m, jnp.int32),
                               jnp.full((16,), e, jnp.int32)])
                    for c8 in range(4):
                        gbufs[b, e, pl.ds(c8 * 16, 16)] = (
                            gbufs[b, e, pl.ds(c8 * 16, 16)] * av)
                scatter_chunk(m, b)
                bn = (b + RD - 2) % RD

                def drain_prev():
                    pltpu.make_async_copy(gbufs.at[bn],
                                          acc_sh.at[db_v.at[m - 2]],
                                          ssems.at[bn]).wait()
                if b >= 2:
                    drain_prev()
                else:
                    pl.when(k > 0)(drain_prev)
                gather_chunk(q, m + RD - 2, bn)
            return carry
        lax.fori_loop(0, NCH // RD, octo, 0)

        pltpu.make_async_copy(gbufs.at[(NCH - 2) % RD],
                              acc_sh.at[db_v.at[NCH - 2]],
                              ssems.at[(NCH - 2) % RD]).wait()
        pltpu.make_async_copy(gbufs.at[(NCH - 1) % RD],
                              acc_sh.at[db_v.at[NCH - 1]],
                              ssems.at[(NCH - 1) % RD]).wait()
        for b0 in range(RD - 2):
            pltpu.make_async_copy(featQ.at[c, q].at[sb_v.at[NCH + b0]],
                                  gbufs.at[(NCH + b0) % RD],
                                  gsems.at[(NCH + b0) % RD]).wait()
        plsc.subcore_barrier()

        pltpu.sync_copy(acc_sh.at[pl.ds(s * STRIPE, STRIPE)],
                        acc_out.at[c, q, pl.ds(s * STRIPE, STRIPE)])
        if q + 1 < Q:
            plsc.subcore_barrier()


def _sc_layer(Q, featQ, eler, srcb, dstb):
    return pl.kernel(
        functools.partial(_sc_layer_body, Q),
        out_type=(
            jax.ShapeDtypeStruct((2, Q, NP, 64), jnp.float32),
            jax.ShapeDtypeStruct((2, NP), jnp.float32),
        ),
        mesh=_mesh,
        compiler_params=pltpu.CompilerParams(needs_layout_passes=False,
                                             use_tc_tiling_on_sc=False),
        scratch_types=[
            pltpu.VMEM((NP,), jnp.float32),
            pltpu.VMEM((NP,), jnp.float32),
            pltpu.VMEM((NCH + RD - 2, CH), jnp.int32),
            pltpu.VMEM((NCH, CH), jnp.int32),
            pltpu.VMEM((NCH, CH), jnp.float32),
            pltpu.VMEM((RD, CH, 64), jnp.float32),
            pltpu.VMEM((STRIPE,), jnp.float32),
            pltpu.SemaphoreType.DMA((RD,)),
            pltpu.SemaphoreType.DMA((RD,)),
            pltpu.VMEM_SHARED((NP,), jnp.float32),
            pltpu.VMEM_SHARED((NP, 64), jnp.float32),
        ],
    )(featQ, eler, srcb, dstb)


TPW = 10320


def _sc_readout_body(uv, aif, bif, o_hbm, u_v, v_v, ai_v, bi_v, o_v):
    c = lax.axis_index("c")
    s = lax.axis_index("s")
    w = c * 16 + s
    pltpu.sync_copy(uv.at[0], u_v)
    pltpu.sync_copy(uv.at[1], v_v)
    pltpu.sync_copy(aif.at[w], ai_v)
    pltpu.sync_copy(bif.at[w], bi_v)

    def body(g, carry):
        ai = ai_v[pl.ds(g * 16, 16)]
        bi = bi_v[pl.ds(g * 16, 16)]
        t = plsc.load_gather(u_v, [ai]) + plsc.load_gather(v_v, [bi])
        o_v[pl.ds(g * 16, 16)] = 1.0 / (1.0 + jnp.exp(-t))
        return carry
    lax.fori_loop(0, TPW // 16, body, 0)
    pltpu.sync_copy(o_v, o_hbm.at[w])


def _sc_readout(uv, aif, bif):
    return pl.kernel(
        _sc_readout_body,
        out_type=jax.ShapeDtypeStruct((32, TPW), jnp.float32),
        mesh=_mesh,
        compiler_params=pltpu.CompilerParams(needs_layout_passes=False),
        scratch_types=[
            pltpu.VMEM((NP,), jnp.float32),
            pltpu.VMEM((NP,), jnp.float32),
            pltpu.VMEM((TPW,), jnp.int32),
            pltpu.VMEM((TPW,), jnp.int32),
            pltpu.VMEM((TPW,), jnp.float32),
        ],
    )(uv, aif, bif)


def _prep_edges(ei):
    src = ei[0].reshape(16, E // 16)
    dst = ei[1].reshape(16, E // 16)
    pad = EPT - E // 16
    src = jnp.pad(src, ((0, 0), (0, pad)))
    dst = jnp.pad(dst, ((0, 0), (0, pad)), constant_values=NP - 1)
    return src, dst


def kernel(x, ei0, ei1, n_pairs, W1_0, al1_0, ar1_0, b1_0, W1_1, al1_1,
           ar1_1, b1_1, W2_0, al2_0, ar2_0, b2_0, W2_1, al2_1, ar2_1, b2_1,
           W_lin, b_lin):
    x_pad = jnp.pad(x, ((0, NP - N), (0, 0)))

    s0, d0 = _prep_edges(ei0)
    s1, d1 = _prep_edges(ei1)
    srcb = jnp.pad(jnp.stack([s0, s1]).reshape(2, 16, NCH, CH),
                   ((0, 0), (0, 0), (0, RD - 2), (0, 0)))
    dstb = jnp.stack([d0, d1]).reshape(2, 16, NCH, CH)

    xQ = jnp.stack([x_pad[:, :64], x_pad[:, 64:]])
    featX = jnp.stack([xQ, xQ])
    eler1 = _tc_logits(x_pad, W1_0, W1_1, al1_0, al1_1, ar1_0, ar1_1)
    acc1, sden1 = _sc_layer(2, featX, eler1, srcb, dstb)
    feat2, eler2 = _tc_mid(acc1, sden1, b1_0, b1_1, W1_0, W1_1, W2_0, W2_1,
                           al2_0, al2_1, ar2_0, ar2_1)
    acc2, sden2 = _sc_layer(2, feat2, eler2, srcb, dstb)
    uv = _tc_out(acc2, sden2, b2_0, b2_1, W_lin, b_lin)

    a_idx = jnp.concatenate([ei0[0], ei1[0], n_pairs[:, 0]])
    b_idx = jnp.concatenate([ei0[1], ei1[1], n_pairs[:, 1]])
    npad = 32 * TPW - a_idx.shape[0]
    aif = jnp.pad(a_idx, (0, npad)).reshape(32, TPW)
    bif = jnp.pad(b_idx, (0, npad)).reshape(32, TPW)

    out = _sc_readout(uv, aif, bif)
    return out.reshape(-1)[: 2 * E + n_pairs.shape[0], None]


# speedup vs baseline: 2.0614x; 1.5208x over previous
import functools

import jax
import jax.numpy as jnp
from jax import lax
from jax.experimental import pallas as pl
from jax.experimental.pallas import tpu as pltpu
from jax.experimental.pallas import tpu_sc as plsc

N = 10000
NP = 10240
F_IN = 128
HID = 256
E = 160000
EPT = 10240
CH = 128
NCH = EPT // CH
STRIPE = NP // 16
RD = 4

_mesh = plsc.VectorSubcoreMesh(core_axis_name="c", subcore_axis_name="s",
                               num_cores=2, num_subcores=16)


def _tc_logits_body(x_ref, w_ref, alar_ref, eler_ref):
    xb = x_ref[...]
    for r in range(2):
        cl = jnp.dot(w_ref[r], alar_ref[r], preferred_element_type=jnp.float32)
        elr = jnp.dot(xb, cl, preferred_element_type=jnp.float32)
        eler_ref[0, r] = elr[:, 0]
        eler_ref[1, r] = elr[:, 1]


def _tc_logits(x, W0, W1, al0, al1, ar0, ar1):
    R = 256
    grid = (NP // R,)
    return pl.pallas_call(
        _tc_logits_body,
        grid=grid,
        in_specs=[
            pl.BlockSpec((R, F_IN), lambda i: (i, 0)),
            pl.BlockSpec((2, F_IN, HID), lambda i: (0, 0, 0)),
            pl.BlockSpec((2, HID, 2), lambda i: (0, 0, 0)),
        ],
        out_specs=pl.BlockSpec((2, 2, R), lambda i: (0, 0, i)),
        out_shape=jax.ShapeDtypeStruct((2, 2, NP), jnp.float32),
    )(x, jnp.stack([W0, W1]),
      jnp.stack([jnp.stack([al0, ar0], 1), jnp.stack([al1, ar1], 1)]))


def _tc_mid_body(*refs):
    a = refs[:4]
    s_ref, b_ref, w1_ref, w2_ref, alar_ref, feat_ref, eler_ref = refs[4:]
    s0 = jnp.maximum(s_ref[0], 1e-30)[:, None]
    s1 = jnp.maximum(s_ref[1], 1e-30)[:, None]
    bm = 0.5 * (b_ref[0] + b_ref[1])
    agg0 = jnp.concatenate([a[0][0, 0], a[1][0, 0]], axis=1) / s0
    agg1 = jnp.concatenate([a[2][0, 0], a[3][0, 0]], axis=1) / s1
    h = 0.5 * (jnp.dot(agg0, w1_ref[0], preferred_element_type=jnp.float32)
               + jnp.dot(agg1, w1_ref[1], preferred_element_type=jnp.float32))
    hb = jax.nn.relu(h + bm[None, :])
    for r in range(2):
        f = jnp.dot(hb, w2_ref[r], preferred_element_type=jnp.float32)
        feat_ref[r, 0] = f[:, :64]
        feat_ref[r, 1] = f[:, 64:]
        elr = jnp.dot(f, alar_ref[r], preferred_element_type=jnp.float32)
        eler_ref[0, r] = elr[:, 0]
        eler_ref[1, r] = elr[:, 1]


def _tc_mid(acc1, s1, b0, b1, W1_0, W1_1, W0, W1, al0, al1, ar0, ar1):
    R = 256
    grid = (NP // R,)
    views = [pl.BlockSpec((1, 1, R, 64), functools.partial(
        lambda i, r, q: (r, q, i, 0), r=r, q=q))
        for r in (0, 1) for q in range(2)]
    return pl.pallas_call(
        _tc_mid_body,
        grid=grid,
        in_specs=views + [
            pl.BlockSpec((2, R), lambda i: (0, i)),
            pl.BlockSpec((2, HID), lambda i: (0, 0)),
            pl.BlockSpec((2, F_IN, HID), lambda i: (0, 0, 0)),
            pl.BlockSpec((2, HID, 128), lambda i: (0, 0, 0)),
            pl.BlockSpec((2, 128, 2), lambda i: (0, 0, 0)),
        ],
        out_specs=[
            pl.BlockSpec((2, 2, R, 64), lambda i: (0, 0, i, 0)),
            pl.BlockSpec((2, 2, R), lambda i: (0, 0, i)),
        ],
        out_shape=[
            jax.ShapeDtypeStruct((2, 2, NP, 64), jnp.float32),
            jax.ShapeDtypeStruct((2, 2, NP), jnp.float32),
        ],
    )(*([acc1] * 4), s1, jnp.stack([b0, b1]), jnp.stack([W1_0, W1_1]),
      jnp.stack([W0, W1]),
      jnp.stack([jnp.stack([al0, ar0], 1), jnp.stack([al1, ar1], 1)]))


def _tc_out_body(a00, a01, a10, a11, s_ref, b_ref, wl_ref, blin_ref, uv_ref):
    s0 = jnp.maximum(s_ref[0], 1e-30)[:, None]
    s1 = jnp.maximum(s_ref[1], 1e-30)[:, None]
    bm = 0.5 * (b_ref[0] + b_ref[1])
    hh = jnp.concatenate(
        [0.5 * (a00[0, 0] / s0 + a10[0, 0] / s1) + bm[None, :64],
         0.5 * (a01[0, 0] / s0 + a11[0, 0] / s1) + bm[None, 64:]], axis=1)
    r = jax.nn.relu(hh)
    uvb = jnp.dot(r, wl_ref[...], preferred_element_type=jnp.float32)
    uv_ref[0] = uvb[:, 0] + blin_ref[0]
    uv_ref[1] = uvb[:, 1]


def _tc_out(acc2, s2, b0, b1, W_lin, b_lin):
    R = 256
    grid = (NP // R,)
    wl = jnp.stack([W_lin[:128, 0], W_lin[128:, 0]], axis=1)
    views = [pl.BlockSpec((1, 1, R, 64), functools.partial(
        lambda i, r, q: (r, q, i, 0), r=r, q=q))
        for r in (0, 1) for q in (0, 1)]
    return pl.pallas_call(
        _tc_out_body,
        grid=grid,
        in_specs=views + [
            pl.BlockSpec((2, R), lambda i: (0, i)),
            pl.BlockSpec((2, 128), lambda i: (0, 0)),
            pl.BlockSpec((128, 2), lambda i: (0, 0)),
            pl.BlockSpec(memory_space=pltpu.SMEM),
        ],
        out_specs=pl.BlockSpec((2, R), lambda i: (0, i)),
        out_shape=jax.ShapeDtypeStruct((2, NP), jnp.float32),
    )(acc2, acc2, acc2, acc2, s2, jnp.stack([b0, b1]), wl, b_lin)


def _sc_layer_body(Q, featQ, eler, srcb, dstb, acc_out, s_out,
                   el_v, er_v, sb_v, db_v, ex_v, gbufs,
                   zs_v, gsems, ssems, s_sh, acc_sh):
    c = lax.axis_index("c")
    s = lax.axis_index("s")
    pltpu.sync_copy(eler.at[0, c], el_v)
    pltpu.sync_copy(eler.at[1, c], er_v)
    pltpu.sync_copy(srcb.at[c, s], sb_v)
    pltpu.sync_copy(dstb.at[c, s], db_v)

    zv = jnp.zeros((16,), jnp.float32)

    def gather_chunk(q, j, b):
        return pltpu.async_copy(featQ.at[c, q].at[sb_v.at[j]], gbufs.at[b],
                                gsems.at[b])

    def scatter_chunk(j, b):
        return pltpu.async_copy(gbufs.at[b], acc_sh.at[db_v.at[j]],
                                ssems.at[b], add=True)

    gather_chunk(0, 0, 0)
    gather_chunk(0, 1, 1)

    def zsrow(r, carry):
        zs_v[pl.ds(r * 16, 16)] = zv
        return carry
    lax.fori_loop(0, STRIPE // 16, zsrow, 0)
    pltpu.sync_copy(zs_v, s_sh.at[pl.ds(s * STRIPE, STRIPE)])

    NG = CH // 16
    @plsc.parallel_loop(0, EPT // 16, unroll=4)
    def att(g):
        si = sb_v[g // NG, pl.ds((g % NG) * 16, 16)]
        di = db_v[g // NG, pl.ds((g % NG) * 16, 16)]
        e = plsc.load_gather(el_v, [si]) + plsc.load_gather(er_v, [di])
        e = jnp.where(e > 0, e, 0.2 * e)
        ex_v[g // NG, pl.ds((g % NG) * 16, 16)] = jnp.exp(e)
    plsc.subcore_barrier()

    def satt(k, carry):
        for b in range(8):
            pltpu.async_copy(ex_v.at[k * 8 + b], s_sh.at[db_v.at[k * 8 + b]],
                             ssems.at[0], add=True)
        for b in range(8):
            pltpu.make_async_copy(ex_v.at[k * 8 + b],
                                  s_sh.at[db_v.at[k * 8 + b]],
                                  ssems.at[0]).wait()
        return carry
    lax.fori_loop(0, NCH // 8, satt, 0)
    plsc.subcore_barrier()

    pltpu.sync_copy(s_sh.at[pl.ds(s * STRIPE, STRIPE)],
                    s_out.at[c, pl.ds(s * STRIPE, STRIPE)])

    for q in range(Q):
        def zgrow(r, carry):
            for c8 in range(4):
                gbufs[RD - 1, r, pl.ds(c8 * 16, 16)] = zv
            return carry
        lax.fori_loop(0, CH, zgrow, 0)
        for k in range(STRIPE // CH):
            pltpu.sync_copy(gbufs.at[RD - 1],
                            acc_sh.at[pl.ds(s * STRIPE + k * CH, CH)])
        plsc.subcore_barrier()

        def quad(k, carry):
            for b in range(RD):
                m = k * RD + b
                pltpu.make_async_copy(featQ.at[c, q].at[sb_v.at[m]],
                                      gbufs.at[b], gsems.at[b]).wait()

                @plsc.parallel_loop(0, CH, unroll=8)
                def scale(e):
                    av = plsc.load_gather(
                        ex_v, [jnp.full((16,), m, jnp.int32),
                               jnp.full((16,), e, jnp.int32)])
                    for c8 in range(4):
                        gbufs[b, e, pl.ds(c8 * 16, 16)] = (
                            gbufs[b, e, pl.ds(c8 * 16, 16)] * av)
                scatter_chunk(m, b)
                bn = (b + RD - 2) % RD

                def drain_prev():
                    pltpu.make_async_copy(gbufs.at[bn],
                                          acc_sh.at[db_v.at[m - 2]],
                                          ssems.at[bn]).wait()
                if b >= 2:
                    drain_prev()
                else:
                    pl.when(k > 0)(drain_prev)
                def prefetch_next():
                    gather_chunk(q, m + RD - 2, bn)
                pl.when(m + RD - 2 < NCH)(prefetch_next)
            return carry
        lax.fori_loop(0, NCH // RD, quad, 0)

        pltpu.make_async_copy(gbufs.at[(NCH - 2) % RD],
                              acc_sh.at[db_v.at[NCH - 2]],
                              ssems.at[(NCH - 2) % RD]).wait()
        pltpu.make_async_copy(gbufs.at[(NCH - 1) % RD],
                              acc_sh.at[db_v.at[NCH - 1]],
                              ssems.at[(NCH - 1) % RD]).wait()
        if q + 1 < Q:
            gather_chunk(q + 1, 0, 0)
            gather_chunk(q + 1, 1, 1)
        plsc.subcore_barrier()

        pltpu.sync_copy(acc_sh.at[pl.ds(s * STRIPE, STRIPE)],
                        acc_out.at[c, q, pl.ds(s * STRIPE, STRIPE)])
        if q + 1 < Q:
            plsc.subcore_barrier()


def _sc_layer(Q, featQ, eler, srcb, dstb):
    return pl.kernel(
        functools.partial(_sc_layer_body, Q),
        out_type=(
            jax.ShapeDtypeStruct((2, Q, NP, 64), jnp.float32),
            jax.ShapeDtypeStruct((2, NP), jnp.float32),
        ),
        mesh=_mesh,
        compiler_params=pltpu.CompilerParams(needs_layout_passes=False,
                                             use_tc_tiling_on_sc=False),
        scratch_types=[
            pltpu.VMEM((NP,), jnp.float32),
            pltpu.VMEM((NP,), jnp.float32),
            pltpu.VMEM((NCH + RD - 2, CH), jnp.int32),
            pltpu.VMEM((NCH, CH), jnp.int32),
            pltpu.VMEM((NCH, CH), jnp.float32),
            pltpu.VMEM((RD, CH, 64), jnp.float32),
            pltpu.VMEM((STRIPE,), jnp.float32),
            pltpu.SemaphoreType.DMA((RD,)),
            pltpu.SemaphoreType.DMA((RD,)),
            pltpu.VMEM_SHARED((NP,), jnp.float32),
            pltpu.VMEM_SHARED((NP, 64), jnp.float32),
        ],
    )(featQ, eler, srcb, dstb)


TPW = 10320


def _sc_readout_body(uv, aif, bif, o_hbm, u_v, v_v, ai_v, bi_v, o_v):
    c = lax.axis_index("c")
    s = lax.axis_index("s")
    w = c * 16 + s
    pltpu.sync_copy(uv.at[0], u_v)
    pltpu.sync_copy(uv.at[1], v_v)
    pltpu.sync_copy(aif.at[w], ai_v)
    pltpu.sync_copy(bif.at[w], bi_v)

    def body(g, carry):
        ai = ai_v[pl.ds(g * 16, 16)]
        bi = bi_v[pl.ds(g * 16, 16)]
        t = plsc.load_gather(u_v, [ai]) + plsc.load_gather(v_v, [bi])
        o_v[pl.ds(g * 16, 16)] = 1.0 / (1.0 + jnp.exp(-t))
        return carry
    lax.fori_loop(0, TPW // 16, body, 0)
    pltpu.sync_copy(o_v, o_hbm.at[w])


def _sc_readout(uv, aif, bif):
    return pl.kernel(
        _sc_readout_body,
        out_type=jax.ShapeDtypeStruct((32, TPW), jnp.float32),
        mesh=_mesh,
        compiler_params=pltpu.CompilerParams(needs_layout_passes=False),
        scratch_types=[
            pltpu.VMEM((NP,), jnp.float32),
            pltpu.VMEM((NP,), jnp.float32),
            pltpu.VMEM((TPW,), jnp.int32),
            pltpu.VMEM((TPW,), jnp.int32),
            pltpu.VMEM((TPW,), jnp.float32),
        ],
    )(uv, aif, bif)


def _prep_edges(ei):
    src = ei[0].reshape(16, E // 16)
    dst = ei[1].reshape(16, E // 16)
    pad = EPT - E // 16
    src = jnp.pad(src, ((0, 0), (0, pad)))
    dst = jnp.pad(dst, ((0, 0), (0, pad)), constant_values=NP - 1)
    return src, dst


def kernel(x, ei0, ei1, n_pairs, W1_0, al1_0, ar1_0, b1_0, W1_1, al1_1,
           ar1_1, b1_1, W2_0, al2_0, ar2_0, b2_0, W2_1, al2_1, ar2_1, b2_1,
           W_lin, b_lin):
    x_pad = jnp.pad(x, ((0, NP - N), (0, 0)))

    s0, d0 = _prep_edges(ei0)
    s1, d1 = _prep_edges(ei1)
    srcb = jnp.pad(jnp.stack([s0, s1]).reshape(2, 16, NCH, CH),
                   ((0, 0), (0, 0), (0, RD - 2), (0, 0)))
    dstb = jnp.stack([d0, d1]).reshape(2, 16, NCH, CH)

    xQ = jnp.stack([x_pad[:, :64], x_pad[:, 64:]])
    featX = jnp.stack([xQ, xQ])
    eler1 = _tc_logits(x_pad, W1_0, W1_1, al1_0, al1_1, ar1_0, ar1_1)
    acc1, sden1 = _sc_layer(2, featX, eler1, srcb, dstb)
    feat2, eler2 = _tc_mid(acc1, sden1, b1_0, b1_1, W1_0, W1_1, W2_0, W2_1,
                           al2_0, al2_1, ar2_0, ar2_1)
    acc2, sden2 = _sc_layer(2, feat2, eler2, srcb, dstb)
    uv = _tc_out(acc2, sden2, b2_0, b2_1, W_lin, b_lin)

    a_idx = jnp.concatenate([ei0[0], ei1[0], n_pairs[:, 0]])
    b_idx = jnp.concatenate([ei0[1], ei1[1], n_pairs[:, 1]])
    npad = 32 * TPW - a_idx.shape[0]
    aif = jnp.pad(a_idx, (0, npad)).reshape(32, TPW)
    bif = jnp.pad(b_idx, (0, npad)).reshape(32, TPW)

    out = _sc_readout(uv, aif, bif)
    return out.reshape(-1)[: 2 * E + n_pairs.shape[0], None]
